# baseline (speedup 1.0000x reference)
import jax
import jax.numpy as jnp
from jax.experimental import pallas as pl
from jax.experimental.pallas import tpu as pltpu

_K = 8192
_D = 32
_BETA = 0.25
_TN = 256
_TW = 4096


def _vq_tc_kernel(z_ref, e_ref, q_ref, loss_ref, e2_ref):
    z = z_ref[...]
    e = e_ref[...]

    @pl.when(pl.program_id(0) == 0)
    def _e2():
        e2_ref[...] = jnp.sum(e * e, axis=1).reshape(1, _K)

    zb = z.astype(jnp.bfloat16).astype(jnp.float32)
    prod2 = jax.lax.dot_general(zb * 2.0, e, (((1,), (1,)), ((), ())),
                                preferred_element_type=jnp.float32)
    z2 = jnp.sum(z * z, axis=1, keepdims=True)
    dist = (z2 + e2_ref[...]) - prod2

    acc_v = None
    acc_i = None
    for t in range(_K // _TW):
        dt = dist[:, t * _TW:(t + 1) * _TW]
        mval = jnp.min(dt, axis=1, keepdims=True)
        ii = jax.lax.broadcasted_iota(jnp.int32, dt.shape, 1)
        lidx = jnp.min(jnp.where(dt <= mval, ii, _TW), axis=1) + t * _TW
        xv = mval[:, 0]
        xvb = xv.astype(jnp.bfloat16).astype(jnp.float32)
        if acc_v is None:
            acc_v, acc_i = xvb, lidx
        else:
            keep = (acc_v < xv) | ((acc_v == xv) & (acc_i < lidx))
            acc_v = jnp.where(keep, acc_v, xvb)
            acc_i = jnp.where(keep, acc_i, lidx)

    kk = jax.lax.broadcasted_iota(jnp.int16, (_TN, _K), 1)
    onehot = (acc_i.astype(jnp.int16)[:, None] == kk).astype(jnp.float32)
    q = jax.lax.dot_general(onehot, e, (((1,), (0,)), ((), ())),
                            preferred_element_type=jnp.float32)
    diff = q - z
    q_ref[...] = z + diff

    @pl.when(pl.program_id(0) == 0)
    def _init():
        loss_ref[...] = jnp.zeros((1, 1), jnp.float32)

    loss_ref[...] += jnp.reshape(jnp.sum(diff * diff), (1, 1))


def kernel(latents, embedding_weight):
    latents_shape = latents.shape
    flat = latents.reshape(-1, _D)
    n = flat.shape[0]
    grid = n // _TN
    q, loss_sum = pl.pallas_call(
        _vq_tc_kernel,
        grid=(grid,),
        in_specs=[
            pl.BlockSpec((_TN, _D), lambda i: (i, 0)),
            pl.BlockSpec((_K, _D), lambda i: (0, 0)),
        ],
        out_specs=[
            pl.BlockSpec((_TN, _D), lambda i: (i, 0)),
            pl.BlockSpec((1, 1), lambda i: (0, 0)),
        ],
        out_shape=[
            jax.ShapeDtypeStruct((n, _D), jnp.float32),
            jax.ShapeDtypeStruct((1, 1), jnp.float32),
        ],
        scratch_shapes=[pltpu.VMEM((1, _K), jnp.float32)],
        compiler_params=pltpu.CompilerParams(
            dimension_semantics=("arbitrary",),
        ),
    )(flat, embedding_weight)
    mse = loss_sum[0, 0] / (n * _D)
    vq_loss = mse * _BETA + mse
    return q.reshape(latents_shape), vq_loss


# speedup vs baseline: 1.0271x; 1.0271x over previous
import jax
import jax.numpy as jnp
from jax.experimental import pallas as pl
from jax.experimental.pallas import tpu as pltpu

_K = 8192
_D = 32
_BETA = 0.25
_TN = 256
_TW = 4096


def _e2_kernel(e_ref, e2_ref):
    e = e_ref[...]
    e2_ref[...] = jnp.sum(e * e, axis=1).reshape(1, _K)


def _vq_tc_kernel(z_ref, e_ref, e2_ref, q_ref, loss_ref):
    z = z_ref[...]
    e = e_ref[...]
    zb = z.astype(jnp.bfloat16).astype(jnp.float32)
    prod2 = jax.lax.dot_general(zb * 2.0, e, (((1,), (1,)), ((), ())),
                                preferred_element_type=jnp.float32)
    z2 = jnp.sum(z * z, axis=1, keepdims=True)
    dist = (z2 + e2_ref[...]) - prod2

    acc_v = None
    acc_i = None
    for t in range(_K // _TW):
        dt = dist[:, t * _TW:(t + 1) * _TW]
        mval = jnp.min(dt, axis=1, keepdims=True)
        ii = jax.lax.broadcasted_iota(jnp.int32, dt.shape, 1)
        lidx = jnp.min(jnp.where(dt <= mval, ii, _TW), axis=1) + t * _TW
        xv = mval[:, 0]
        xvb = xv.astype(jnp.bfloat16).astype(jnp.float32)
        if acc_v is None:
            acc_v, acc_i = xvb, lidx
        else:
            keep = (acc_v < xv) | ((acc_v == xv) & (acc_i < lidx))
            acc_v = jnp.where(keep, acc_v, xvb)
            acc_i = jnp.where(keep, acc_i, lidx)

    kk = jax.lax.broadcasted_iota(jnp.int16, (_TN, _K), 1)
    onehot = (acc_i.astype(jnp.int16)[:, None] == kk).astype(jnp.float32)
    q = jax.lax.dot_general(onehot, e, (((1,), (0,)), ((), ())),
                            preferred_element_type=jnp.float32)
    diff = q - z
    q_ref[...] = z + diff

    @pl.when(pl.program_id(0) == 0)
    def _init():
        loss_ref[...] = jnp.zeros((1, 1), jnp.float32)

    loss_ref[...] += jnp.reshape(jnp.sum(diff * diff), (1, 1))


def kernel(latents, embedding_weight):
    latents_shape = latents.shape
    flat = latents.reshape(-1, _D)
    n = flat.shape[0]
    grid = n // _TN
    e2 = pl.pallas_call(
        _e2_kernel,
        out_shape=jax.ShapeDtypeStruct((1, _K), jnp.float32),
    )(embedding_weight)
    q, loss_sum = pl.pallas_call(
        _vq_tc_kernel,
        grid=(grid,),
        in_specs=[
            pl.BlockSpec((_TN, _D), lambda i: (i, 0)),
            pl.BlockSpec((_K, _D), lambda i: (0, 0)),
            pl.BlockSpec((1, _K), lambda i: (0, 0)),
        ],
        out_specs=[
            pl.BlockSpec((_TN, _D), lambda i: (i, 0)),
            pl.BlockSpec((1, 1), lambda i: (0, 0)),
        ],
        out_shape=[
            jax.ShapeDtypeStruct((n, _D), jnp.float32),
            jax.ShapeDtypeStruct((1, 1), jnp.float32),
        ],
        compiler_params=pltpu.CompilerParams(
            dimension_semantics=("arbitrary",),
        ),
    )(flat, embedding_weight, e2)
    mse = loss_sum[0, 0] / (n * _D)
    vq_loss = mse * _BETA + mse
    return q.reshape(latents_shape), vq_loss


# speedup vs baseline: 1.1721x; 1.1412x over previous
import functools

import jax
import jax.numpy as jnp
from jax import lax
from jax.experimental import pallas as pl
from jax.experimental.pallas import tpu as pltpu
from jax.experimental.pallas import tpu_sc as plsc

_K = 8192
_D = 32
_BETA = 0.25
_TN = 256
_TW = 4096
_RPL = 128 // _D


def _vq_argmin_kernel(z_ref, e_ref, idxl_ref, idxc_ref, loss_ref):
    z = z_ref[...]
    e = e_ref[...]
    zb = z.astype(jnp.bfloat16).astype(jnp.float32)
    prod = jax.lax.dot_general(zb, e, (((1,), (1,)), ((), ())),
                               preferred_element_type=jnp.float32)
    z2 = jnp.sum(z * z, axis=1, keepdims=True)
    e2 = jnp.sum(e * e, axis=1)
    dist = (z2 + e2[None, :]) - 2.0 * prod

    acc_v = None
    acc_t = None
    acc_i = None
    for t in range(_K // _TW):
        dt = dist[:, t * _TW:(t + 1) * _TW]
        mval = jnp.min(dt, axis=1, keepdims=True)
        ii = jax.lax.broadcasted_iota(jnp.int32, dt.shape, 1) + t * _TW
        lidx = jnp.min(jnp.where(dt <= mval, ii, _K), axis=1)
        xv = mval[:, 0]
        xvb = xv.astype(jnp.bfloat16).astype(jnp.float32)
        if acc_v is None:
            acc_v, acc_t, acc_i = xvb, xv, lidx
        else:
            keep = (acc_v < xv) | ((acc_v == xv) & (acc_i < lidx))
            acc_v = jnp.where(keep, acc_v, xvb)
            acc_t = jnp.where(keep, acc_t, xv)
            acc_i = jnp.where(keep, acc_i, lidx)

    idxl_ref[...] = (acc_i // _RPL).reshape(idxl_ref.shape)
    idxc_ref[...] = acc_i[:, None]

    @pl.when(pl.program_id(0) == 0)
    def _init():
        loss_ref[...] = jnp.zeros((1, 1), jnp.float32)

    loss_ref[...] += jnp.reshape(jnp.sum(acc_t), (1, 1))


_sc_info = plsc.get_sparse_core_info()
_NW = _sc_info.num_cores * _sc_info.num_subcores


def _sc_gather(table_lines, idx_lines):
    n = idx_lines.shape[0]
    b_per_w = n // _NW
    mesh = plsc.VectorSubcoreMesh(core_axis_name="c", subcore_axis_name="s")

    @functools.partial(
        pl.kernel, mesh=mesh,
        out_type=jax.ShapeDtypeStruct((n, 128), jnp.float32),
        scratch_types=[
            pltpu.VMEM((b_per_w,), jnp.int32),
            pltpu.VMEM((b_per_w, 128), jnp.float32),
            pltpu.SemaphoreType.DMA,
        ],
    )
    def k(table_hbm, idx_hbm, out_hbm, idx_v, rows_v, sem):
        wid = lax.axis_index("s") * _sc_info.num_cores + lax.axis_index("c")
        base = wid * b_per_w
        pltpu.sync_copy(idx_hbm.at[pl.ds(base, b_per_w)], idx_v)
        pltpu.async_copy(table_hbm.at[idx_v], rows_v, sem).wait()
        pltpu.sync_copy(rows_v, out_hbm.at[pl.ds(base, b_per_w)])

    return k(table_lines, idx_lines)


def _st_kernel(z_ref, rows_ref, idxc_ref, out_ref):
    z = z_ref[...]
    rows = rows_ref[...]
    off = idxc_ref[...] % _RPL
    q = rows[:, 0:_D]
    for oi in range(1, _RPL):
        q = jnp.where(off == oi, rows[:, oi * _D:(oi + 1) * _D], q)
    out_ref[...] = z + (q - z)


def kernel(latents, embedding_weight):
    latents_shape = latents.shape
    flat = latents.reshape(-1, _D)
    n = flat.shape[0]
    grid = n // _TN
    idx_lines, idx_col, loss_sum = pl.pallas_call(
        _vq_argmin_kernel,
        grid=(grid,),
        in_specs=[
            pl.BlockSpec((_TN, _D), lambda i: (i, 0)),
            pl.BlockSpec((_K, _D), lambda i: (0, 0)),
        ],
        out_specs=[
            pl.BlockSpec((1, 1, _TN), lambda i: (i, 0, 0)),
            pl.BlockSpec((_TN, 1), lambda i: (i, 0)),
            pl.BlockSpec((1, 1), lambda i: (0, 0)),
        ],
        out_shape=[
            jax.ShapeDtypeStruct((grid, 1, _TN), jnp.int32),
            jax.ShapeDtypeStruct((n, 1), jnp.int32),
            jax.ShapeDtypeStruct((1, 1), jnp.float32),
        ],
        compiler_params=pltpu.CompilerParams(
            dimension_semantics=("arbitrary",),
        ),
    )(flat, embedding_weight)

    table_lines = embedding_weight.reshape(_K // _RPL, 128)
    rows = _sc_gather(table_lines, idx_lines.reshape(-1))

    out = pl.pallas_call(
        _st_kernel,
        grid=(grid,),
        in_specs=[
            pl.BlockSpec((_TN, _D), lambda i: (i, 0)),
            pl.BlockSpec((_TN, 128), lambda i: (i, 0)),
            pl.BlockSpec((_TN, 1), lambda i: (i, 0)),
        ],
        out_specs=pl.BlockSpec((_TN, _D), lambda i: (i, 0)),
        out_shape=jax.ShapeDtypeStruct((n, _D), jnp.float32),
    )(flat, rows, idx_col)

    mse = loss_sum[0, 0] / (n * _D)
    vq_loss = mse * _BETA + mse
    return out.reshape(latents_shape), vq_loss
